# baseline (speedup 1.0000x reference)
import functools

import jax
import jax.numpy as jnp
from jax import lax
from jax.experimental import pallas as pl
from jax.experimental.pallas import tpu as pltpu
from jax.experimental.pallas import tpu_sc as plsc

BATCH = 16384
DIM = 128
TABLE_ROWS = 1000

NUM_CORES = 2
NUM_SUBCORES = 16
NUM_WORKERS = NUM_CORES * NUM_SUBCORES
ROWS_PER_WORKER = BATCH // NUM_WORKERS
CHUNK = 64
NUM_CHUNKS = ROWS_PER_WORKER // CHUNK

STAGE_ROWS = 64
STAGE_LAST = TABLE_ROWS - 15 * STAGE_ROWS


def _mlp_body(table_ref, w1_ref, b1_ref, w2_ref, b2_ref, out_ref):
    x = table_ref[...]
    h = jnp.dot(x, w1_ref[...], preferred_element_type=jnp.float32) + b1_ref[...]
    h = h * jax.nn.sigmoid(h)
    h = jnp.dot(h, w2_ref[...], preferred_element_type=jnp.float32) + b2_ref[...]
    out_ref[...] = h * jax.nn.sigmoid(h)


def _mlp_table(table, W1, b1, W2, b2):
    return pl.pallas_call(
        _mlp_body,
        out_shape=jax.ShapeDtypeStruct((TABLE_ROWS, DIM), jnp.float32),
    )(table, W1, b1, W2, b2)


_SC_MESH = plsc.VectorSubcoreMesh(core_axis_name="c", subcore_axis_name="s")


@functools.partial(
    pl.kernel,
    mesh=_SC_MESH,
    out_type=jax.ShapeDtypeStruct((BATCH, DIM), jnp.float32),
    scratch_types=[
        pltpu.VMEM_SHARED((TABLE_ROWS, DIM), jnp.float32),
        pltpu.VMEM((NUM_CHUNKS, CHUNK), jnp.int32),
        pltpu.VMEM((ROWS_PER_WORKER, DIM), jnp.float32),
        [pltpu.SemaphoreType.DMA] * NUM_CHUNKS,
        pltpu.SemaphoreType.DMA,
    ],
)
def _gather(table_hbm, idx_hbm, out_hbm, table_sh, idx_v, rows_v, gsems, wsem):
    cid = lax.axis_index("c")
    sid = lax.axis_index("s")
    wid = sid * NUM_CORES + cid
    base = wid * ROWS_PER_WORKER

    @pl.when(sid < 15)
    def _stage_full():
        pltpu.sync_copy(
            table_hbm.at[pl.ds(sid * STAGE_ROWS, STAGE_ROWS)],
            table_sh.at[pl.ds(sid * STAGE_ROWS, STAGE_ROWS)],
        )

    @pl.when(sid == 15)
    def _stage_tail():
        pltpu.sync_copy(
            table_hbm.at[pl.ds(15 * STAGE_ROWS, STAGE_LAST)],
            table_sh.at[pl.ds(15 * STAGE_ROWS, STAGE_LAST)],
        )

    pltpu.sync_copy(idx_hbm.at[wid], idx_v)
    plsc.subcore_barrier()
    gathers = []
    for j in range(NUM_CHUNKS):
        gathers.append(
            pltpu.async_copy(
                table_sh.at[idx_v.at[j]],
                rows_v.at[pl.ds(j * CHUNK, CHUNK)],
                gsems[j],
            )
        )
    writes = []
    for j in range(NUM_CHUNKS):
        gathers[j].wait()
        writes.append(
            pltpu.async_copy(
                rows_v.at[pl.ds(j * CHUNK, CHUNK)],
                out_hbm.at[pl.ds(base + j * CHUNK, CHUNK)],
                wsem,
            )
        )
    for w in writes:
        w.wait()


def kernel(diffusion_step, embedding, W1, b1, W2, b2):
    transformed = _mlp_table(
        embedding, W1, b1.reshape(1, DIM), W2, b2.reshape(1, DIM)
    )
    idx = diffusion_step.astype(jnp.int32).reshape(NUM_WORKERS, NUM_CHUNKS, CHUNK)
    return _gather(transformed, idx)


# speedup vs baseline: 1.0809x; 1.0809x over previous
import functools

import jax
import jax.numpy as jnp
from jax import lax
from jax.experimental import pallas as pl
from jax.experimental.pallas import tpu as pltpu
from jax.experimental.pallas import tpu_sc as plsc

BATCH = 16384
DIM = 128
TABLE_ROWS = 1000

NUM_CORES = 2
NUM_SUBCORES = 16
NUM_WORKERS = NUM_CORES * NUM_SUBCORES
ROWS_PER_WORKER = BATCH // NUM_WORKERS
CHUNK = 64
NUM_CHUNKS = ROWS_PER_WORKER // CHUNK

STAGE_ROWS = 64
STAGE_LAST = TABLE_ROWS - 15 * STAGE_ROWS


def _mlp_body(table_ref, w1_ref, b1_ref, w2_ref, b2_ref, out_ref):
    x = table_ref[...]
    h = jnp.dot(x, w1_ref[...], preferred_element_type=jnp.float32) + b1_ref[...]
    h = h * jax.nn.sigmoid(h)
    h = jnp.dot(h, w2_ref[...], preferred_element_type=jnp.float32) + b2_ref[...]
    out_ref[...] = h * jax.nn.sigmoid(h)


def _mlp_table(table, W1, b1, W2, b2):
    return pl.pallas_call(
        _mlp_body,
        out_shape=jax.ShapeDtypeStruct((TABLE_ROWS, DIM), jnp.float32),
    )(table, W1, b1, W2, b2)


_SC_MESH = plsc.VectorSubcoreMesh(core_axis_name="c", subcore_axis_name="s")


@functools.partial(
    pl.kernel,
    mesh=_SC_MESH,
    out_type=jax.ShapeDtypeStruct((BATCH, DIM), jnp.float32),
    scratch_types=[
        pltpu.VMEM_SHARED((TABLE_ROWS, DIM), jnp.float32),
        pltpu.VMEM((ROWS_PER_WORKER,), jnp.int32),
        pltpu.VMEM((ROWS_PER_WORKER, DIM), jnp.float32),
        [pltpu.SemaphoreType.DMA] * NUM_CHUNKS,
        pltpu.SemaphoreType.DMA,
        pltpu.SemaphoreType.DMA,
    ],
)
def _gather(table_hbm, idx_hbm, out_hbm, table_sh, idx_v, rows_v, gsems, wsem, ssem):
    cid = lax.axis_index("c")
    sid = lax.axis_index("s")
    wid = sid * NUM_CORES + cid
    base = wid * ROWS_PER_WORKER

    idx_cp = pltpu.async_copy(idx_hbm.at[pl.ds(base, ROWS_PER_WORKER)], idx_v, wsem)

    @pl.when(sid < 15)
    def _stage_full():
        pltpu.async_copy(
            table_hbm.at[pl.ds(sid * STAGE_ROWS, STAGE_ROWS)],
            table_sh.at[pl.ds(sid * STAGE_ROWS, STAGE_ROWS)],
            ssem,
        ).wait()

    @pl.when(sid == 15)
    def _stage_tail():
        pltpu.async_copy(
            table_hbm.at[pl.ds(15 * STAGE_ROWS, STAGE_LAST)],
            table_sh.at[pl.ds(15 * STAGE_ROWS, STAGE_LAST)],
            ssem,
        ).wait()

    idx_cp.wait()
    plsc.subcore_barrier()
    gathers = []
    for j in range(NUM_CHUNKS):
        gathers.append(
            pltpu.async_copy(
                table_sh.at[idx_v.at[pl.ds(j * CHUNK, CHUNK)]],
                rows_v.at[pl.ds(j * CHUNK, CHUNK)],
                gsems[j],
            )
        )
    writes = []
    for j in range(NUM_CHUNKS):
        gathers[j].wait()
        writes.append(
            pltpu.async_copy(
                rows_v.at[pl.ds(j * CHUNK, CHUNK)],
                out_hbm.at[pl.ds(base + j * CHUNK, CHUNK)],
                wsem,
            )
        )
    for w in writes:
        w.wait()


def kernel(diffusion_step, embedding, W1, b1, W2, b2):
    transformed = _mlp_table(
        embedding, W1, b1.reshape(1, DIM), W2, b2.reshape(1, DIM)
    )
    return _gather(transformed, diffusion_step.astype(jnp.int32))
